# baseline (speedup 1.0000x reference)
import functools

import jax
import jax.numpy as jnp
from jax import lax
from jax.experimental import pallas as pl
from jax.experimental.pallas import tpu as pltpu
from jax.experimental.pallas import tpu_sc as plsc

NC = 2
NS = 16
NW = NC * NS
IPR = 128
IDXCH = 1024
W = 128


@functools.lru_cache(maxsize=None)
def _make_agg(n_nodes, e_pad):
    total_chunks = e_pad // IDXCH
    cpt = total_chunks // NS
    idx_rows = IDXCH // IPR
    buf_rows = IPR
    slab = (-(-(n_nodes + 1) // NS) + 7) // 8 * 8
    np_rows = NS * slab
    zr = slab
    mesh = plsc.VectorSubcoreMesh(core_axis_name="c", subcore_axis_name="s")

    @functools.partial(
        pl.kernel,
        out_type=jax.ShapeDtypeStruct((np_rows, W), jnp.float32),
        mesh=mesh,
        scratch_types=[
            pltpu.VMEM((idx_rows, IPR), jnp.int32),
            pltpu.VMEM((idx_rows, IPR), jnp.int32),
            pltpu.VMEM((buf_rows, W), jnp.float32),
            pltpu.VMEM((buf_rows, W), jnp.float32),
            pltpu.VMEM_SHARED((np_rows, W), jnp.float32),
            pltpu.SemaphoreType.DMA,
            pltpu.SemaphoreType.DMA,
            pltpu.SemaphoreType.DMA,
            pltpu.SemaphoreType.DMA,
        ],
    )
    def agg(src_hbm, dst_hbm, y_hbm, out_hbm, srcv, dstv,
            buf0, buf1, acc, gs0, gs1, ss0, ss1):
        bufs = (buf0, buf1)
        gsems = (gs0, gs1)
        ssems = (ss0, ss1)
        rows = buf0
        c = lax.axis_index("c")
        s = lax.axis_index("s")

        @pl.when(c == 0)
        def _work():
            zvec = jnp.zeros((16,), jnp.float32)

            def zrow(i, carry):
                for k in range(W // 16):
                    rows[i, pl.ds(k * 16, 16)] = zvec
                return carry

            lax.fori_loop(0, min(buf_rows, zr), zrow, 0)
            r0 = s * zr
            off = 0
            while off < zr:
                step = min(buf_rows, zr - off)
                pltpu.sync_copy(rows.at[pl.ds(0, step)],
                                acc.at[pl.ds(r0 + off, step)])
                off += step
            plsc.subcore_barrier()

            row_base = s * cpt * idx_rows

            def chunk(i, carry):
                rb = row_base + i * idx_rows
                pltpu.sync_copy(src_hbm.at[pl.ds(rb, idx_rows)], srcv)
                pltpu.sync_copy(dst_hbm.at[pl.ds(rb, idx_rows)], dstv)
                gcp = {}
                scp = {}
                gcp[0] = pltpu.async_copy(y_hbm.at[srcv.at[0]], bufs[0],
                                          gsems[0])
                gcp[1] = pltpu.async_copy(y_hbm.at[srcv.at[1]], bufs[1],
                                          gsems[1])
                for u in range(idx_rows):
                    b = u % 2
                    gcp[u].wait()
                    scp[u] = pltpu.async_copy(bufs[b], acc.at[dstv.at[u]],
                                              ssems[b], add=True)
                    if u + 2 < idx_rows:
                        scp[u].wait()
                        gcp[u + 2] = pltpu.async_copy(
                            y_hbm.at[srcv.at[u + 2]], bufs[b], gsems[b])
                scp[idx_rows - 2].wait()
                scp[idx_rows - 1].wait()
                return carry

            lax.fori_loop(0, cpt, chunk, 0)
            plsc.subcore_barrier()

            pltpu.sync_copy(acc.at[pl.ds(r0, zr)], out_hbm.at[pl.ds(r0, zr)])

    return agg


def _prep_body(x_ref, wcat_ref, y_ref, r_ref):
    n = y_ref.shape[0]
    hid = r_ref.shape[1]
    out = lax.dot_general(x_ref[...], wcat_ref[...],
                          (((1,), (0,)), ((), ())),
                          preferred_element_type=jnp.float32)
    col = lax.broadcasted_iota(jnp.int32, (n, W), 1)
    y_ref[...] = out[:, :W] + jnp.where(col == hid, 1.0, 0.0)
    r_ref[...] = out[:, W:]


def _mid_body(p_ref, r_ref, b1_ref, g1_ref, be1_ref, w2_ref,
              yr2_ref, ci_ref):
    hid = r_ref.shape[1]
    ssum = p_ref[:, :hid]
    cnt = p_ref[:, hid:hid + 1]
    cclip = jnp.maximum(cnt, 1.0)
    h = ssum / cclip + b1_ref[...][None, :] + r_ref[...]
    mu = jnp.mean(h, axis=0, keepdims=True)
    var = jnp.mean((h - mu) ** 2, axis=0, keepdims=True)
    hn = jnp.maximum(
        g1_ref[...][None, :] * (h - mu) / jnp.sqrt(var + 1e-5)
        + be1_ref[...][None, :], 0.0)
    yr2_ref[...] = lax.dot_general(hn, w2_ref[...], (((1,), (0,)), ((), ())),
                                   preferred_element_type=jnp.float32)
    ci_ref[...] = cclip


def _fin_body(p_ref, yr2_ref, ci_ref, b2_ref, g2_ref, be2_ref, o_ref):
    hid = o_ref.shape[1]
    ssum = p_ref[:, :hid]
    r2 = yr2_ref[...][:, hid:]
    h = ssum / ci_ref[...] + b2_ref[...][None, :] + r2
    mu = jnp.mean(h, axis=0, keepdims=True)
    var = jnp.mean((h - mu) ** 2, axis=0, keepdims=True)
    hn = jnp.maximum(
        g2_ref[...][None, :] * (h - mu) / jnp.sqrt(var + 1e-5)
        + be2_ref[...][None, :], 0.0)
    o_ref[...] = jnp.max(hn, axis=0, keepdims=True)


def kernel(x, edge_index, Wl1, Wr1, b1, Wl2, Wr2, b2, g1, beta1, g2, beta2):
    n, in_dim = x.shape
    hid = Wl1.shape[0]
    e = edge_index.shape[1]

    e_pad = -(-e // (NW * IDXCH)) * (NW * IDXCH)
    pad = e_pad - e
    src = jnp.concatenate(
        [edge_index[0], jnp.zeros((pad,), jnp.int32)]).reshape(-1, IPR)
    dst = jnp.concatenate(
        [edge_index[1], jnp.full((pad,), n, jnp.int32)]).reshape(-1, IPR)

    w1cat = jnp.concatenate(
        [Wl1.T, jnp.zeros((in_dim, W - hid), jnp.float32), Wr1.T], axis=1)
    yaug, r1 = pl.pallas_call(
        _prep_body,
        out_shape=[jax.ShapeDtypeStruct((n, W), jnp.float32),
                   jax.ShapeDtypeStruct((n, hid), jnp.float32)],
    )(x, w1cat)

    part1 = _make_agg(n, e_pad)(src, dst, yaug)[:n, :]

    w2cat = jnp.concatenate([Wl2.T, Wr2.T], axis=1)
    yr2, ci = pl.pallas_call(
        _mid_body,
        out_shape=[jax.ShapeDtypeStruct((n, W), jnp.float32),
                   jax.ShapeDtypeStruct((n, 1), jnp.float32)],
    )(part1, r1, b1, g1, beta1, w2cat)

    part2 = _make_agg(n, e_pad)(src, dst, yr2)[:n, :]

    o = pl.pallas_call(
        _fin_body,
        out_shape=jax.ShapeDtypeStruct((1, hid), jnp.float32),
    )(part2, yr2, ci, b2, g2, beta2)
    return o.reshape((hid,))


# speedup vs baseline: 1.3744x; 1.3744x over previous
import functools

import jax
import jax.numpy as jnp
from jax import lax
from jax.experimental import pallas as pl
from jax.experimental.pallas import tpu as pltpu
from jax.experimental.pallas import tpu_sc as plsc

NC = 2
NS = 16
NW = NC * NS
IPR = 128
IDXCH = 1024
W = 128


@functools.lru_cache(maxsize=None)
def _make_agg(n_nodes, e_pad):
    total_chunks = e_pad // IDXCH
    cpt = total_chunks // NW
    idx_rows = IDXCH // IPR
    buf_rows = IPR
    slab = (-(-(n_nodes + 1) // NS) + 7) // 8 * 8
    np_rows = NS * slab
    zr = slab
    mesh = plsc.VectorSubcoreMesh(core_axis_name="c", subcore_axis_name="s")

    @functools.partial(
        pl.kernel,
        out_type=jax.ShapeDtypeStruct((NC * np_rows, W), jnp.float32),
        mesh=mesh,
        scratch_types=[
            pltpu.VMEM((idx_rows, IPR), jnp.int32),
            pltpu.VMEM((idx_rows, IPR), jnp.int32),
            pltpu.VMEM((buf_rows, W), jnp.float32),
            pltpu.VMEM((buf_rows, W), jnp.float32),
            pltpu.VMEM_SHARED((np_rows, W), jnp.float32),
            pltpu.SemaphoreType.DMA,
            pltpu.SemaphoreType.DMA,
            pltpu.SemaphoreType.DMA,
            pltpu.SemaphoreType.DMA,
        ],
    )
    def agg(src_hbm, dst_hbm, y_hbm, out_hbm, srcv, dstv,
            buf0, buf1, acc, gs0, gs1, ss0, ss1):
        bufs = (buf0, buf1)
        gsems = (gs0, gs1)
        ssems = (ss0, ss1)
        rows = buf0
        c = lax.axis_index("c")
        s = lax.axis_index("s")

        zvec = jnp.zeros((16,), jnp.float32)

        def zrow(i, carry):
            for k in range(W // 16):
                rows[i, pl.ds(k * 16, 16)] = zvec
            return carry

        lax.fori_loop(0, min(buf_rows, zr), zrow, 0)
        r0 = s * zr
        off = 0
        while off < zr:
            step = min(buf_rows, zr - off)
            pltpu.sync_copy(rows.at[pl.ds(0, step)],
                            acc.at[pl.ds(r0 + off, step)])
            off += step
        plsc.subcore_barrier()

        row_base = (c * NS + s) * cpt * idx_rows

        def chunk(i, carry):
            rb = row_base + i * idx_rows
            pltpu.sync_copy(src_hbm.at[pl.ds(rb, idx_rows)], srcv)
            pltpu.sync_copy(dst_hbm.at[pl.ds(rb, idx_rows)], dstv)
            gcp = {}
            scp = {}
            gcp[0] = pltpu.async_copy(y_hbm.at[srcv.at[0]], bufs[0],
                                      gsems[0])
            gcp[1] = pltpu.async_copy(y_hbm.at[srcv.at[1]], bufs[1],
                                      gsems[1])
            for u in range(idx_rows):
                b = u % 2
                gcp[u].wait()
                scp[u] = pltpu.async_copy(bufs[b], acc.at[dstv.at[u]],
                                          ssems[b], add=True)
                if u + 2 < idx_rows:
                    scp[u].wait()
                    gcp[u + 2] = pltpu.async_copy(
                        y_hbm.at[srcv.at[u + 2]], bufs[b], gsems[b])
            scp[idx_rows - 2].wait()
            scp[idx_rows - 1].wait()
            return carry

        lax.fori_loop(0, cpt, chunk, 0)
        plsc.subcore_barrier()

        pltpu.sync_copy(acc.at[pl.ds(r0, zr)],
                        out_hbm.at[pl.ds(c * np_rows + r0, zr)])

    return agg, np_rows


def _prep_body(x_ref, wcat_ref, y_ref, r_ref):
    n = y_ref.shape[0]
    hid = r_ref.shape[1]
    out = lax.dot_general(x_ref[...], wcat_ref[...],
                          (((1,), (0,)), ((), ())),
                          preferred_element_type=jnp.float32)
    col = lax.broadcasted_iota(jnp.int32, (n, W), 1)
    y_ref[...] = out[:, :W] + jnp.where(col == hid, 1.0, 0.0)
    r_ref[...] = out[:, W:]


def _mid_body(pa_ref, pb_ref, r_ref, b1_ref, g1_ref, be1_ref, w2_ref,
              yr2_ref, ci_ref):
    hid = r_ref.shape[1]
    p = pa_ref[...] + pb_ref[...]
    ssum = p[:, :hid]
    cnt = p[:, hid:hid + 1]
    cclip = jnp.maximum(cnt, 1.0)
    h = ssum / cclip + b1_ref[...][None, :] + r_ref[...]
    mu = jnp.mean(h, axis=0, keepdims=True)
    var = jnp.mean((h - mu) ** 2, axis=0, keepdims=True)
    hn = jnp.maximum(
        g1_ref[...][None, :] * (h - mu) / jnp.sqrt(var + 1e-5)
        + be1_ref[...][None, :], 0.0)
    yr2_ref[...] = lax.dot_general(hn, w2_ref[...], (((1,), (0,)), ((), ())),
                                   preferred_element_type=jnp.float32)
    ci_ref[...] = cclip


def _fin_body(pa_ref, pb_ref, yr2_ref, ci_ref, b2_ref, g2_ref, be2_ref,
              o_ref):
    hid = o_ref.shape[1]
    ssum = pa_ref[:, :hid] + pb_ref[:, :hid]
    r2 = yr2_ref[...][:, hid:]
    h = ssum / ci_ref[...] + b2_ref[...][None, :] + r2
    mu = jnp.mean(h, axis=0, keepdims=True)
    var = jnp.mean((h - mu) ** 2, axis=0, keepdims=True)
    hn = jnp.maximum(
        g2_ref[...][None, :] * (h - mu) / jnp.sqrt(var + 1e-5)
        + be2_ref[...][None, :], 0.0)
    o_ref[...] = jnp.max(hn, axis=0, keepdims=True)


def kernel(x, edge_index, Wl1, Wr1, b1, Wl2, Wr2, b2, g1, beta1, g2, beta2):
    n, in_dim = x.shape
    hid = Wl1.shape[0]
    e = edge_index.shape[1]

    e_pad = -(-e // (NW * IDXCH)) * (NW * IDXCH)
    pad = e_pad - e
    src = jnp.concatenate(
        [edge_index[0], jnp.zeros((pad,), jnp.int32)]).reshape(-1, IPR)
    dst = jnp.concatenate(
        [edge_index[1], jnp.full((pad,), n, jnp.int32)]).reshape(-1, IPR)

    w1cat = jnp.concatenate(
        [Wl1.T, jnp.zeros((in_dim, W - hid), jnp.float32), Wr1.T], axis=1)
    yaug, r1 = pl.pallas_call(
        _prep_body,
        out_shape=[jax.ShapeDtypeStruct((n, W), jnp.float32),
                   jax.ShapeDtypeStruct((n, hid), jnp.float32)],
    )(x, w1cat)

    agg, np_rows = _make_agg(n, e_pad)
    part1 = agg(src, dst, yaug)
    p1a = lax.slice(part1, (0, 0), (n, W))
    p1b = lax.slice(part1, (np_rows, 0), (np_rows + n, W))

    w2cat = jnp.concatenate([Wl2.T, Wr2.T], axis=1)
    yr2, ci = pl.pallas_call(
        _mid_body,
        out_shape=[jax.ShapeDtypeStruct((n, W), jnp.float32),
                   jax.ShapeDtypeStruct((n, 1), jnp.float32)],
    )(p1a, p1b, r1, b1, g1, beta1, w2cat)

    part2 = agg(src, dst, yr2)
    p2a = lax.slice(part2, (0, 0), (n, W))
    p2b = lax.slice(part2, (np_rows, 0), (np_rows + n, W))

    o = pl.pallas_call(
        _fin_body,
        out_shape=jax.ShapeDtypeStruct((1, hid), jnp.float32),
    )(p2a, p2b, yr2, ci, b2, g2, beta2)
    return o.reshape((hid,))
